# baseline (speedup 1.0000x reference)
import functools

import jax
import jax.numpy as jnp
from jax import lax
from jax.experimental import pallas as pl
from jax.experimental.pallas import tpu as pltpu
from jax.experimental.pallas import tpu_sc as plsc

N = 10000
E = 320000
NC = 2
NS = 16
L = 16
NW = NC * NS
EPW = E // NW
NCH = EPW // L
NZB = N // L
EPAD = EPW + 128 - (EPW % 128)

def _worker_id():
    return lax.axis_index("s") * NC + lax.axis_index("c")


def _zero_vmem(acc_v):
    zeros = jnp.zeros((L,), jnp.float32)

    @plsc.parallel_loop(0, NZB, unroll=5)
    def _(i):
        acc_v[pl.ds(i * L, L)] = zeros


@functools.lru_cache(maxsize=None)
def _sc_kernels():
    mesh = plsc.VectorSubcoreMesh(
        core_axis_name="c", subcore_axis_name="s", num_cores=NC, num_subcores=NS
    )

    @functools.partial(
        pl.kernel,
        out_type=jax.ShapeDtypeStruct((NW, N), jnp.float32),
        mesh=mesh,
        compiler_params=pltpu.CompilerParams(needs_layout_passes=False),
        scratch_types=[
            pltpu.VMEM((2, EPAD), jnp.int32),
            pltpu.VMEM((N,), jnp.float32),
            pltpu.SemaphoreType.DMA,
        ],
    )
    def _sc_degree(ei_hbm, out_hbm, ei_v, acc_v, sem):
        wid = _worker_id()
        start = wid * EPW
        start_al = (start // 128) * 128
        off = start - start_al
        cp = pltpu.async_copy(ei_hbm.at[:, pl.ds(start_al, EPAD)], ei_v, sem)
        _zero_vmem(acc_v)
        cp.wait()
        ones = jnp.ones((L,), jnp.float32)

        @plsc.parallel_loop(0, NCH, unroll=25)
        def _(i):
            d_idx = ei_v[1, pl.ds(off + i * L, L)]
            plsc.addupdate_scatter(acc_v, [d_idx], ones)
        pltpu.sync_copy(acc_v, out_hbm.at[wid])

    @functools.partial(
        pl.kernel,
        out_type=jax.ShapeDtypeStruct((NW, N), jnp.float32),
        mesh=mesh,
        compiler_params=pltpu.CompilerParams(needs_layout_passes=False),
        scratch_types=[
            pltpu.VMEM((2, EPAD), jnp.int32),
            pltpu.VMEM((N,), jnp.float32),
            pltpu.VMEM((N,), jnp.float32),
            pltpu.SemaphoreType.DMA,
            pltpu.SemaphoreType.DMA,
        ],
    )
    def _sc_scatter(ei_hbm, g_hbm, out_hbm, ei_v, g_v, acc_v, sem1, sem2):
        wid = _worker_id()
        start = wid * EPW
        start_al = (start // 128) * 128
        off = start - start_al
        cp1 = pltpu.async_copy(ei_hbm.at[:, pl.ds(start_al, EPAD)], ei_v, sem1)
        cp2 = pltpu.async_copy(g_hbm, g_v, sem2)
        _zero_vmem(acc_v)
        cp1.wait()
        cp2.wait()

        @plsc.parallel_loop(0, NCH, unroll=25)
        def _(i):
            sl = pl.ds(off + i * L, L)
            s_idx = ei_v[0, sl]
            d_idx = ei_v[1, sl]
            vals = plsc.load_gather(g_v, [s_idx])
            plsc.addupdate_scatter(acc_v, [d_idx], vals)
        pltpu.sync_copy(acc_v, out_hbm.at[wid])

    return _sc_degree, _sc_scatter


XBK = 1024


def _tc_p(x_ref, wz_ref, wa_ref, bz_ref, p_ref, c_ref, w_s):
    i = pl.program_id(0)

    @pl.when(i == 0)
    def _():
        w_s[...] = jnp.sum(wz_ref[...] * wa_ref[...], axis=1)[None, :]
        c_ref[...] = jnp.sum(bz_ref[...] * wa_ref[...], keepdims=True)

    p_ref[...] = jnp.sum(x_ref[...] * w_s[...], axis=1)


def _tc_g1(degp_ref, p_ref, dinv_ref, g1_ref):
    deg = jnp.sum(degp_ref[...], axis=0) + 1.0
    dinv = lax.rsqrt(deg)
    dinv_ref[...] = dinv
    g1_ref[...] = dinv * p_ref[...]


def _tc_mid(t1p_ref, g1_ref, dinv_ref, c_ref, g2_ref):
    t1 = jnp.sum(t1p_ref[...], axis=0) + g1_ref[...]
    q = dinv_ref[...] * t1 + c_ref[0, 0]
    g2_ref[...] = dinv_ref[...] * q


def _tc_final(t2p_ref, g2_ref, dinv_ref, ba_ref, mu_ref, lv_ref, dist_ref, out_ref):
    t2 = jnp.sum(t2p_ref[...], axis=0) + g2_ref[...]
    a = dinv_ref[...] * t2 + ba_ref[0, 0]
    m = jnp.max(a)
    e = jnp.exp(a - m)
    s = jnp.sum(e)
    er = e[None, :]
    um = jnp.dot(er, mu_ref[...], preferred_element_type=jnp.float32)
    uv = jnp.dot(er, jnp.exp(lv_ref[...]), preferred_element_type=jnp.float32)
    out_ref[...] = (um + uv * dist_ref[...]) / s


def kernel(x, edge_index, dist, W_z, b_z, W_a, b_a, mu, log_var):
    f32 = jnp.float32
    ei = edge_index.astype(jnp.int32)
    wa2 = W_a.reshape(1, W_a.shape[0]).astype(f32)
    bz2 = b_z.reshape(1, b_z.shape[0]).astype(f32)
    ba2 = b_a.reshape(1, 1).astype(f32)

    sc_degree, sc_scatter = _sc_kernels()
    degp = sc_degree(ei)

    p, c = pl.pallas_call(
        _tc_p,
        grid=(pl.cdiv(N, XBK),),
        in_specs=[
            pl.BlockSpec((XBK, x.shape[1]), lambda i: (i, 0)),
            pl.BlockSpec(W_z.shape, lambda i: (0, 0)),
            pl.BlockSpec(wa2.shape, lambda i: (0, 0)),
            pl.BlockSpec(bz2.shape, lambda i: (0, 0)),
        ],
        out_specs=[
            pl.BlockSpec((XBK,), lambda i: (i,)),
            pl.BlockSpec((1, 1), lambda i: (0, 0)),
        ],
        out_shape=[
            jax.ShapeDtypeStruct((N,), f32),
            jax.ShapeDtypeStruct((1, 1), f32),
        ],
        scratch_shapes=[pltpu.VMEM((1, x.shape[1]), f32)],
    )(x, W_z, wa2, bz2)

    dinv, g1 = pl.pallas_call(
        _tc_g1,
        out_shape=[
            jax.ShapeDtypeStruct((N,), f32),
            jax.ShapeDtypeStruct((N,), f32),
        ],
    )(degp, p)

    t1p = sc_scatter(ei, g1)

    g2 = pl.pallas_call(
        _tc_mid,
        out_shape=jax.ShapeDtypeStruct((N,), f32),
    )(t1p, g1, dinv, c)

    t2p = sc_scatter(ei, g2)

    out = pl.pallas_call(
        _tc_final,
        out_shape=jax.ShapeDtypeStruct((dist.shape[0], dist.shape[1]), f32),
    )(t2p, g2, dinv, ba2, mu, log_var, dist)

    return out


# speedup vs baseline: 1.0724x; 1.0724x over previous
import functools

import jax
import jax.numpy as jnp
from jax import lax
from jax.experimental import pallas as pl
from jax.experimental.pallas import tpu as pltpu
from jax.experimental.pallas import tpu_sc as plsc

N = 10000
E = 320000
NC = 2
NS = 16
L = 16
NW = NC * NS
EPW = E // NW
NCH = EPW // L
NZB = N // L
EPAD = EPW + 128 - (EPW % 128)

def _worker_id():
    return lax.axis_index("s") * NC + lax.axis_index("c")


def _zero_vmem(acc_v):
    zeros = jnp.zeros((L,), jnp.float32)

    @plsc.parallel_loop(0, NZB, unroll=5)
    def _(i):
        acc_v[pl.ds(i * L, L)] = zeros


@functools.lru_cache(maxsize=None)
def _sc_kernels():
    mesh = plsc.VectorSubcoreMesh(
        core_axis_name="c", subcore_axis_name="s", num_cores=NC, num_subcores=NS
    )

    @functools.partial(
        pl.kernel,
        out_type=jax.ShapeDtypeStruct((NW, N), jnp.float32),
        mesh=mesh,
        compiler_params=pltpu.CompilerParams(needs_layout_passes=False),
        scratch_types=[
            pltpu.VMEM((2, EPAD), jnp.int32),
            pltpu.VMEM((N,), jnp.float32),
            pltpu.SemaphoreType.DMA,
        ],
    )
    def _sc_degree(ei_hbm, out_hbm, ei_v, acc_v, sem):
        wid = _worker_id()
        start = wid * EPW
        start_al = (start // 128) * 128
        off = start - start_al
        cp = pltpu.async_copy(ei_hbm.at[:, pl.ds(start_al, EPAD)], ei_v, sem)
        _zero_vmem(acc_v)
        cp.wait()
        ones = jnp.ones((L,), jnp.float32)

        @plsc.parallel_loop(0, NCH, unroll=5)
        def _(i):
            d_idx = ei_v[1, pl.ds(off + i * L, L)]
            plsc.addupdate_scatter(acc_v, [d_idx], ones)
        pltpu.sync_copy(acc_v, out_hbm.at[wid])

    @functools.partial(
        pl.kernel,
        out_type=jax.ShapeDtypeStruct((NW, N), jnp.float32),
        mesh=mesh,
        compiler_params=pltpu.CompilerParams(needs_layout_passes=False),
        scratch_types=[
            pltpu.VMEM((2, EPAD), jnp.int32),
            pltpu.VMEM((N,), jnp.float32),
            pltpu.VMEM((N,), jnp.float32),
            pltpu.SemaphoreType.DMA,
            pltpu.SemaphoreType.DMA,
        ],
    )
    def _sc_scatter(ei_hbm, g_hbm, out_hbm, ei_v, g_v, acc_v, sem1, sem2):
        wid = _worker_id()
        start = wid * EPW
        start_al = (start // 128) * 128
        off = start - start_al
        cp1 = pltpu.async_copy(ei_hbm.at[:, pl.ds(start_al, EPAD)], ei_v, sem1)
        cp2 = pltpu.async_copy(g_hbm, g_v, sem2)
        _zero_vmem(acc_v)
        cp1.wait()
        cp2.wait()

        @plsc.parallel_loop(0, NCH, unroll=5)
        def _(i):
            sl = pl.ds(off + i * L, L)
            s_idx = ei_v[0, sl]
            d_idx = ei_v[1, sl]
            vals = plsc.load_gather(g_v, [s_idx])
            plsc.addupdate_scatter(acc_v, [d_idx], vals)
        pltpu.sync_copy(acc_v, out_hbm.at[wid])

    return _sc_degree, _sc_scatter


def _tc_p(x_ref, wz_ref, wa_ref, bz_ref, p_ref, c_ref):
    w = jnp.sum(wz_ref[...] * wa_ref[...], axis=1)
    p_ref[...] = jnp.sum(x_ref[...] * w[None, :], axis=1)
    c_ref[...] = jnp.sum(bz_ref[...] * wa_ref[...], keepdims=True)


def _tc_g1(degp_ref, p_ref, dinv_ref, g1_ref):
    deg = jnp.sum(degp_ref[...], axis=0) + 1.0
    dinv = lax.rsqrt(deg)
    dinv_ref[...] = dinv
    g1_ref[...] = dinv * p_ref[...]


def _tc_mid(t1p_ref, g1_ref, dinv_ref, c_ref, g2_ref):
    t1 = jnp.sum(t1p_ref[...], axis=0) + g1_ref[...]
    q = dinv_ref[...] * t1 + c_ref[0, 0]
    g2_ref[...] = dinv_ref[...] * q


def _tc_final(t2p_ref, g2_ref, dinv_ref, ba_ref, mu_ref, lv_ref, dist_ref, out_ref):
    t2 = jnp.sum(t2p_ref[...], axis=0) + g2_ref[...]
    a = dinv_ref[...] * t2 + ba_ref[0, 0]
    m = jnp.max(a)
    e = jnp.exp(a - m)
    s = jnp.sum(e)
    er = e[None, :]
    um = jnp.dot(er, mu_ref[...], preferred_element_type=jnp.float32)
    uv = jnp.dot(er, jnp.exp(lv_ref[...]), preferred_element_type=jnp.float32)
    out_ref[...] = (um + uv * dist_ref[...]) / s


def kernel(x, edge_index, dist, W_z, b_z, W_a, b_a, mu, log_var):
    f32 = jnp.float32
    ei = edge_index.astype(jnp.int32)
    wa2 = W_a.reshape(1, W_a.shape[0]).astype(f32)
    bz2 = b_z.reshape(1, b_z.shape[0]).astype(f32)
    ba2 = b_a.reshape(1, 1).astype(f32)

    sc_degree, sc_scatter = _sc_kernels()
    degp = sc_degree(ei)

    p, c = pl.pallas_call(
        _tc_p,
        out_shape=[
            jax.ShapeDtypeStruct((N,), f32),
            jax.ShapeDtypeStruct((1, 1), f32),
        ],
    )(x, W_z, wa2, bz2)

    dinv, g1 = pl.pallas_call(
        _tc_g1,
        out_shape=[
            jax.ShapeDtypeStruct((N,), f32),
            jax.ShapeDtypeStruct((N,), f32),
        ],
    )(degp, p)

    t1p = sc_scatter(ei, g1)

    g2 = pl.pallas_call(
        _tc_mid,
        out_shape=jax.ShapeDtypeStruct((N,), f32),
    )(t1p, g1, dinv, c)

    t2p = sc_scatter(ei, g2)

    out = pl.pallas_call(
        _tc_final,
        out_shape=jax.ShapeDtypeStruct((dist.shape[0], dist.shape[1]), f32),
    )(t2p, g2, dinv, ba2, mu, log_var, dist)

    return out
